# Initial kernel scaffold; baseline (speedup 1.0000x reference)
import functools

import jax
import jax.numpy as jnp
from jax.experimental import pallas as pl


def _finish_kernel(feat_ref, denom_ref, b_ref, skip_ref, o_ref, *, do_elu, heads):
    feat = feat_ref[...]
    denom = denom_ref[...]
    d = jnp.repeat(denom, 128 // heads, axis=1)
    out = feat / (d + 1e-16) + b_ref[...]
    if do_elu:
        out = jnp.where(out > 0, out, jnp.expm1(out))
    o_ref[...] = out + skip_ref[...]


def _finish(feat, denom, b, skip, *, do_elu, heads):
    n = feat.shape[0]
    blk = 128
    grid = (n // blk,)
    return pl.pallas_call(
        functools.partial(_finish_kernel, do_elu=do_elu, heads=heads),
        grid=grid,
        in_specs=[
            pl.BlockSpec((blk, 128), lambda i: (i, 0)),
            pl.BlockSpec((blk, heads), lambda i: (i, 0)),
            pl.BlockSpec((1, 128), lambda i: (0, 0)),
            pl.BlockSpec((blk, 128), lambda i: (i, 0)),
        ],
        out_specs=pl.BlockSpec((blk, 128), lambda i: (i, 0)),
        out_shape=jax.ShapeDtypeStruct((n, 128), jnp.float32),
    )(feat, denom, b.reshape(1, 128), skip)


def _gat_layer(x, src, dst, W, b, att_src, att_dst, heads, out_ch, do_elu):
    N = x.shape[0]
    h = (x @ W.T).reshape(N, heads, out_ch)
    a_src = (h * att_src).sum(-1)
    a_dst = (h * att_dst).sum(-1)
    alpha = a_src[src] + a_dst[dst]
    alpha = jnp.maximum(alpha, 0.2 * alpha)
    e = jnp.exp(alpha)
    denom = jax.ops.segment_sum(e, dst, num_segments=N)
    hs = h[src]
    num = jax.ops.segment_sum(hs * e[:, :, None], dst, num_segments=N)
    feat = num.reshape(N, heads * out_ch)
    return _finish(feat, denom, b, x, do_elu=do_elu, heads=heads)


def kernel(x, edge_index, W1, b1, att_src1, att_dst1, W2, b2, att_src2, att_dst2):
    N = x.shape[0]
    loop = jnp.arange(N, dtype=edge_index.dtype)
    src = jnp.concatenate([edge_index[0], loop])
    dst = jnp.concatenate([edge_index[1], loop])
    npad = 10112
    xp = jnp.pad(x, ((0, npad - N), (0, 0)))
    h = _gat_layer(xp, src, dst, W1, b1, att_src1, att_dst1, 8, 16, True)
    h = _gat_layer(h, src, dst, W2, b2, att_src2, att_dst2, 1, 128, False)
    return h[:N]


# speedup vs baseline: 1.1275x; 1.1275x over previous
import functools

import jax
import jax.numpy as jnp
from jax.experimental import pallas as pl


def _finish_kernel(feat_ref, denom_ref, b_ref, skip_ref, o_ref, *, do_elu, heads):
    feat = feat_ref[...]
    denom = denom_ref[...]
    d = jnp.repeat(denom, 128 // heads, axis=1)
    out = feat / (d + 1e-16) + b_ref[...]
    if do_elu:
        out = jnp.where(out > 0, out, jnp.exp(jnp.minimum(out, 0.0)) - 1.0)
    o_ref[...] = out + skip_ref[...]


def _finish(feat, denom, b, skip, *, do_elu, heads):
    n = feat.shape[0]
    blk = 128
    grid = (n // blk,)
    return pl.pallas_call(
        functools.partial(_finish_kernel, do_elu=do_elu, heads=heads),
        grid=grid,
        in_specs=[
            pl.BlockSpec((blk, 128), lambda i: (i, 0)),
            pl.BlockSpec((blk, heads), lambda i: (i, 0)),
            pl.BlockSpec((1, 128), lambda i: (0, 0)),
            pl.BlockSpec((blk, 128), lambda i: (i, 0)),
        ],
        out_specs=pl.BlockSpec((blk, 128), lambda i: (i, 0)),
        out_shape=jax.ShapeDtypeStruct((n, 128), jnp.float32),
    )(feat, denom, b.reshape(1, 128), skip)


def _gat_layer(x, src, dst, W, b, att_src, att_dst, heads, out_ch, do_elu):
    N = x.shape[0]
    h = (x @ W.T).reshape(N, heads, out_ch)
    a_src = (h * att_src).sum(-1)
    a_dst = (h * att_dst).sum(-1)
    alpha = a_src[src] + a_dst[dst]
    alpha = jnp.maximum(alpha, 0.2 * alpha)
    e = jnp.exp(alpha)
    denom = jax.ops.segment_sum(e, dst, num_segments=N)
    hs = h[src]
    num = jax.ops.segment_sum(hs * e[:, :, None], dst, num_segments=N)
    feat = num.reshape(N, heads * out_ch)
    return _finish(feat, denom, b, x, do_elu=do_elu, heads=heads)


def kernel(x, edge_index, W1, b1, att_src1, att_dst1, W2, b2, att_src2, att_dst2):
    N = x.shape[0]
    loop = jnp.arange(N, dtype=edge_index.dtype)
    src = jnp.concatenate([edge_index[0], loop])
    dst = jnp.concatenate([edge_index[1], loop])
    npad = 10112
    xp = jnp.pad(x, ((0, npad - N), (0, 0)))
    h = _gat_layer(xp, src, dst, W1, b1, att_src1, att_dst1, 8, 16, True)
    h = _gat_layer(h, src, dst, W2, b2, att_src2, att_dst2, 1, 128, False)
    return h[:N]


# speedup vs baseline: 25.5060x; 22.6221x over previous
import functools

import jax
import jax.numpy as jnp
from jax import lax
from jax.experimental import pallas as pl
from jax.experimental.pallas import tpu as pltpu
from jax.experimental.pallas import tpu_sc as plsc

NPAD = 10112
EPAD = 331776
PER_TILE = EPAD // 32
K = 128
CHUNKS = PER_TILE // K
ROWS_PER_TILE = NPAD // 16


def _prep_kernel(x_ref, wt_ref, gs_ref, gd_ref, h_ref, as_ref, ad_ref):
    h = jnp.dot(x_ref[...], wt_ref[...], preferred_element_type=jnp.float32)
    h_ref[...] = h
    as_ref[...] = jnp.dot(h, gs_ref[...], preferred_element_type=jnp.float32)
    ad_ref[...] = jnp.dot(h, gd_ref[...], preferred_element_type=jnp.float32)


def _prep(xp, W, att_src, att_dst, heads, out_ch):
    eye = jnp.repeat(jnp.eye(heads, 16, dtype=jnp.float32), out_ch, axis=0)
    gs = eye * att_src.reshape(128, 1)
    gd = eye * att_dst.reshape(128, 1)
    blk = 128
    return pl.pallas_call(
        _prep_kernel,
        grid=(NPAD // blk,),
        in_specs=[
            pl.BlockSpec((blk, 128), lambda i: (i, 0)),
            pl.BlockSpec((128, 128), lambda i: (0, 0)),
            pl.BlockSpec((128, 16), lambda i: (0, 0)),
            pl.BlockSpec((128, 16), lambda i: (0, 0)),
        ],
        out_specs=[
            pl.BlockSpec((blk, 128), lambda i: (i, 0)),
            pl.BlockSpec((blk, 16), lambda i: (i, 0)),
            pl.BlockSpec((blk, 16), lambda i: (i, 0)),
        ],
        out_shape=[
            jax.ShapeDtypeStruct((NPAD, 128), jnp.float32),
            jax.ShapeDtypeStruct((NPAD, 16), jnp.float32),
            jax.ShapeDtypeStruct((NPAD, 16), jnp.float32),
        ],
    )(xp, W.T, gs, gd)


def _sc_edge_kernel(heads):
    mesh = plsc.VectorSubcoreMesh(core_axis_name="c", subcore_axis_name="s")

    @functools.partial(
        pl.kernel,
        out_type=jax.ShapeDtypeStruct((2, NPAD, 144), jnp.float32),
        mesh=mesh,
        compiler_params=pltpu.CompilerParams(use_tc_tiling_on_sc=False),
        scratch_types=[
            pltpu.VMEM((K,), jnp.int32),
            pltpu.VMEM((K,), jnp.int32),
            pltpu.VMEM((K, 16), jnp.float32),
            pltpu.VMEM((K, 16), jnp.float32),
            pltpu.VMEM((K, 128), jnp.float32),
            pltpu.VMEM((K, 144), jnp.float32),
            pltpu.VMEM_SHARED((NPAD, 144), jnp.float32),
            pltpu.SemaphoreType.DMA,
            pltpu.SemaphoreType.DMA,
            pltpu.SemaphoreType.DMA,
        ],
    )
    def k(feat_hbm, as_hbm, ad_hbm, src_hbm, dst_hbm, zero_hbm, out_hbm,
          src_v, dst_v, as_v, ad_v, feat_v, row_v, acc_sh, sem0, sem1, sem2):
        c = lax.axis_index("c")
        s = lax.axis_index("s")
        w = c * 16 + s
        pltpu.sync_copy(zero_hbm.at[pl.ds(s * ROWS_PER_TILE, ROWS_PER_TILE)],
                        acc_sh.at[pl.ds(s * ROWS_PER_TILE, ROWS_PER_TILE)])
        plsc.subcore_barrier()

        @pl.loop(0, CHUNKS)
        def _chunk(g):
            off = g * K
            pltpu.sync_copy(src_hbm.at[w, pl.ds(off, K)], src_v)
            pltpu.sync_copy(dst_hbm.at[w, pl.ds(off, K)], dst_v)
            cp0 = pltpu.async_copy(as_hbm.at[src_v], as_v, sem0)
            cp1 = pltpu.async_copy(ad_hbm.at[dst_v], ad_v, sem1)
            cp2 = pltpu.async_copy(feat_hbm.at[src_v], feat_v, sem2)
            cp0.wait()
            cp1.wait()
            cp2.wait()

            @pl.loop(0, K)
            def _edge(e):
                va = as_v[e, :] + ad_v[e, :]
                va = jnp.maximum(va, 0.2 * va)
                wv = jnp.exp(va)
                row_v[e, pl.ds(128, 16)] = wv
                for j in range(8):
                    lane = j if heads == 8 else 0
                    wj = wv.at[jnp.full((16,), lane, jnp.int32)].get(
                        mode="promise_in_bounds")
                    row_v[e, pl.ds(16 * j, 16)] = (
                        feat_v[e, pl.ds(16 * j, 16)] * wj)

            pltpu.sync_copy(row_v, acc_sh.at[dst_v], add=True)

        plsc.subcore_barrier()
        pltpu.sync_copy(acc_sh.at[pl.ds(s * ROWS_PER_TILE, ROWS_PER_TILE)],
                        out_hbm.at[c, pl.ds(s * ROWS_PER_TILE, ROWS_PER_TILE)])

    return k


def _finish_kernel(f0_ref, f1_ref, d0_ref, d1_ref, b_ref, skip_ref, o_ref,
                   *, do_elu, heads):
    feat = f0_ref[...] + f1_ref[...]
    denom = d0_ref[...] + d1_ref[...]
    d = jnp.repeat(denom, 128 // heads, axis=1)
    out = feat / (d + 1e-16) + b_ref[...]
    if do_elu:
        out = jnp.where(out > 0, out, jnp.exp(jnp.minimum(out, 0.0)) - 1.0)
    o_ref[...] = out + skip_ref[...]


def _finish(f0, f1, d0, d1, b, skip, *, do_elu, heads):
    blk = 128
    return pl.pallas_call(
        functools.partial(_finish_kernel, do_elu=do_elu, heads=heads),
        grid=(NPAD // blk,),
        in_specs=[
            pl.BlockSpec((blk, 128), lambda i: (i, 0)),
            pl.BlockSpec((blk, 128), lambda i: (i, 0)),
            pl.BlockSpec((blk, heads), lambda i: (i, 0)),
            pl.BlockSpec((blk, heads), lambda i: (i, 0)),
            pl.BlockSpec((1, 128), lambda i: (0, 0)),
            pl.BlockSpec((blk, 128), lambda i: (i, 0)),
        ],
        out_specs=pl.BlockSpec((blk, 128), lambda i: (i, 0)),
        out_shape=jax.ShapeDtypeStruct((NPAD, 128), jnp.float32),
    )(f0, f1, d0, d1, b.reshape(1, 128), skip)


def _gat_layer(xp, srcm, dstm, zero, W, b, att_src, att_dst, heads, out_ch,
               do_elu):
    feat, a_s, a_d = _prep(xp, W, att_src, att_dst, heads, out_ch)
    acc = _sc_edge_kernel(heads)(feat, a_s, a_d, srcm, dstm, zero)
    f0 = acc[0, :, :128]
    f1 = acc[1, :, :128]
    d0 = acc[0, :, 128:128 + heads]
    d1 = acc[1, :, 128:128 + heads]
    return _finish(f0, f1, d0, d1, b, xp, do_elu=do_elu, heads=heads)


def kernel(x, edge_index, W1, b1, att_src1, att_dst1, W2, b2, att_src2,
           att_dst2):
    N = x.shape[0]
    loop = jnp.arange(N, dtype=jnp.int32)
    src = jnp.concatenate([edge_index[0].astype(jnp.int32), loop])
    dst = jnp.concatenate([edge_index[1].astype(jnp.int32), loop])
    npad_e = EPAD - src.shape[0]
    src = jnp.concatenate([src, jnp.full((npad_e,), N, jnp.int32)])
    dst = jnp.concatenate([dst, jnp.full((npad_e,), N, jnp.int32)])
    srcm = src.reshape(32, PER_TILE)
    dstm = dst.reshape(32, PER_TILE)
    zero = jnp.zeros((NPAD, 144), jnp.float32)
    xp = jnp.pad(x, ((0, NPAD - N), (0, 0)))
    h = _gat_layer(xp, srcm, dstm, zero, W1, b1, att_src1, att_dst1, 8, 16,
                   True)
    h = _gat_layer(h, srcm, dstm, zero, W2, b2, att_src2, att_dst2, 1, 128,
                   False)
    return h[:N]


# speedup vs baseline: 33.6080x; 1.3177x over previous
import functools

import jax
import jax.numpy as jnp
from jax import lax
from jax.experimental import pallas as pl
from jax.experimental.pallas import tpu as pltpu
from jax.experimental.pallas import tpu_sc as plsc

NPAD = 10112
K = 64
IB = 2
CHUNKS = 164
NBLK = CHUNKS // IB
PER_TILE = CHUNKS * K
EPAD = 32 * PER_TILE
ROWS_PER_TILE = NPAD // 16


def _prep_kernel(x_ref, wt_ref, gs_ref, gd_ref, cat_ref, ad_ref):
    h = jnp.dot(x_ref[...], wt_ref[...], preferred_element_type=jnp.float32)
    cat_ref[:, 16:144] = h
    cat_ref[:, 0:16] = jnp.dot(h, gs_ref[...],
                               preferred_element_type=jnp.float32)
    ad_ref[...] = jnp.dot(h, gd_ref[...], preferred_element_type=jnp.float32)


def _prep(xp, W, att_src, att_dst, heads, out_ch):
    eye = jnp.repeat(jnp.eye(heads, 16, dtype=jnp.float32), out_ch, axis=0)
    gs = eye * att_src.reshape(128, 1)
    gd = eye * att_dst.reshape(128, 1)
    blk = 128
    return pl.pallas_call(
        _prep_kernel,
        grid=(NPAD // blk,),
        in_specs=[
            pl.BlockSpec((blk, 128), lambda i: (i, 0)),
            pl.BlockSpec((128, 128), lambda i: (0, 0)),
            pl.BlockSpec((128, 16), lambda i: (0, 0)),
            pl.BlockSpec((128, 16), lambda i: (0, 0)),
        ],
        out_specs=[
            pl.BlockSpec((blk, 144), lambda i: (i, 0)),
            pl.BlockSpec((blk, 16), lambda i: (i, 0)),
        ],
        out_shape=[
            jax.ShapeDtypeStruct((NPAD, 144), jnp.float32),
            jax.ShapeDtypeStruct((NPAD, 16), jnp.float32),
        ],
    )(xp, W.T, gs, gd)


def _sc_edge_kernel(heads):
    mesh = plsc.VectorSubcoreMesh(core_axis_name="c", subcore_axis_name="s")

    @functools.partial(
        pl.kernel,
        out_type=jax.ShapeDtypeStruct((2, NPAD, 144), jnp.float32),
        mesh=mesh,
        compiler_params=pltpu.CompilerParams(use_tc_tiling_on_sc=False),
        scratch_types=[
            pltpu.VMEM((IB, K), jnp.int32),
            pltpu.VMEM((IB, K), jnp.int32),
            pltpu.VMEM((IB, K), jnp.int32),
            pltpu.VMEM((IB, K), jnp.int32),
            pltpu.VMEM((K, 144), jnp.float32),
            pltpu.VMEM((K, 144), jnp.float32),
            pltpu.VMEM((K, 16), jnp.float32),
            pltpu.VMEM((K, 16), jnp.float32),
            pltpu.VMEM((K, 144), jnp.float32),
            pltpu.VMEM((K, 144), jnp.float32),
            pltpu.VMEM((K,), jnp.int32),
            pltpu.VMEM((K,), jnp.int32),
            pltpu.VMEM_SHARED((NPAD, 144), jnp.float32),
            pltpu.SemaphoreType.DMA,
            pltpu.SemaphoreType.DMA,
            pltpu.SemaphoreType.DMA,
            pltpu.SemaphoreType.DMA,
            pltpu.SemaphoreType.DMA,
            pltpu.SemaphoreType.DMA,
            pltpu.SemaphoreType.DMA,
            pltpu.SemaphoreType.DMA,
        ],
    )
    def k(cat_hbm, ad_hbm, src_hbm, dst_hbm, zero_hbm, out_hbm,
          si0, si1, di0, di1, ct0, ct1, ad0, ad1, rw0, rw1, sd0, sd1, acc_sh,
          ix0, ix1, cs0, cs1, as0, as1, ss0, ss1):
        c = lax.axis_index("c")
        s = lax.axis_index("s")
        w = c * 16 + s
        si = (si0, si1)
        di = (di0, di1)
        ct = (ct0, ct1)
        ad = (ad0, ad1)
        rw = (rw0, rw1)
        sd = (sd0, sd1)
        ixs = (ix0, ix1)
        css = (cs0, cs1)
        ass = (as0, as1)
        sss = (ss0, ss1)

        def idx_start(blk, ib):
            pltpu.make_async_copy(src_hbm.at[w, blk], si[ib], ixs[ib]).start()
            pltpu.make_async_copy(dst_hbm.at[w, blk], di[ib], ixs[ib]).start()

        def idx_wait(blk, ib):
            pltpu.make_async_copy(src_hbm.at[w, blk], si[ib], ixs[ib]).wait()
            pltpu.make_async_copy(dst_hbm.at[w, blk], di[ib], ixs[ib]).wait()

        def gath_start(ib, r, b):
            pltpu.make_async_copy(cat_hbm.at[si[ib].at[r]], ct[b],
                                  css[b]).start()
            pltpu.make_async_copy(ad_hbm.at[di[ib].at[r]], ad[b],
                                  ass[b]).start()

        def gath_wait(ib, r, b):
            pltpu.make_async_copy(cat_hbm.at[si[ib].at[r]], ct[b],
                                  css[b]).wait()
            pltpu.make_async_copy(ad_hbm.at[di[ib].at[r]], ad[b],
                                  ass[b]).wait()

        def scat_start(b):
            pltpu.make_async_copy(rw[b], acc_sh.at[sd[b]],
                                  sss[b]).start(add=True)

        def scat_wait(b):
            pltpu.make_async_copy(rw[b], acc_sh.at[sd[b]], sss[b]).wait()

        def compute(b):
            ct_v, ad_v, rw_v = ct[b], ad[b], rw[b]

            @pl.loop(0, K, unroll=4)
            def _edge(e):
                va = ct_v[e, pl.ds(0, 16)] + ad_v[e, :]
                va = jnp.maximum(va, 0.2 * va)
                wv = jnp.exp(va)
                rw_v[e, pl.ds(128, 16)] = wv
                for j in range(8):
                    lane = j if heads == 8 else 0
                    wj = wv.at[jnp.full((16,), lane, jnp.int32)].get(
                        mode="promise_in_bounds")
                    rw_v[e, pl.ds(16 * j, 16)] = (
                        ct_v[e, pl.ds(16 + 16 * j, 16)] * wj)

        pltpu.sync_copy(zero_hbm.at[pl.ds(s * ROWS_PER_TILE, ROWS_PER_TILE)],
                        acc_sh.at[pl.ds(s * ROWS_PER_TILE, ROWS_PER_TILE)])
        plsc.subcore_barrier()

        idx_start(0, 0)
        idx_start(1, 1)
        idx_wait(0, 0)
        gath_start(0, 0, 0)

        @pl.loop(0, CHUNKS, step=4)
        def _chunk(g4):
            for u in range(4):
                g = g4 + u
                ib = (u // IB) % 2
                b = u % 2
                nib = ((u + 1) // IB) % 2
                nb = (u + 1) % 2

                gath_wait(ib, b, b)

                @pl.when(g >= 2)
                def _():
                    scat_wait(b)

                for t in range(K // 16):
                    sd[b][pl.ds(16 * t, 16)] = di[ib][b, pl.ds(16 * t, 16)]

                if u % 2 == 1:
                    @pl.when(g // IB + 2 < NBLK)
                    def _():
                        pltpu.make_async_copy(
                            src_hbm.at[w, g // IB + 2], si[ib],
                            ixs[ib]).start()
                        pltpu.make_async_copy(
                            dst_hbm.at[w, g // IB + 2], di[ib],
                            ixs[ib]).start()

                    @pl.when(g + 1 < CHUNKS)
                    def _():
                        pltpu.make_async_copy(
                            src_hbm.at[w, (g + 1) // IB], si[nib],
                            ixs[nib]).wait()
                        pltpu.make_async_copy(
                            dst_hbm.at[w, (g + 1) // IB], di[nib],
                            ixs[nib]).wait()

                @pl.when(g + 1 < CHUNKS)
                def _():
                    gath_start(nib, nb, nb)

                compute(b)
                scat_start(b)

        scat_wait(0)
        scat_wait(1)
        plsc.subcore_barrier()
        pltpu.sync_copy(acc_sh.at[pl.ds(s * ROWS_PER_TILE, ROWS_PER_TILE)],
                        out_hbm.at[c, pl.ds(s * ROWS_PER_TILE, ROWS_PER_TILE)])

    return k


def _finish_kernel(f0_ref, f1_ref, d0_ref, d1_ref, b_ref, skip_ref, o_ref,
                   *, do_elu, heads):
    feat = f0_ref[...] + f1_ref[...]
    denom = d0_ref[...] + d1_ref[...]
    d = jnp.repeat(denom, 128 // heads, axis=1)
    out = feat / (d + 1e-16) + b_ref[...]
    if do_elu:
        out = jnp.where(out > 0, out, jnp.exp(jnp.minimum(out, 0.0)) - 1.0)
    o_ref[...] = out + skip_ref[...]


def _finish(f0, f1, d0, d1, b, skip, *, do_elu, heads):
    blk = 128
    return pl.pallas_call(
        functools.partial(_finish_kernel, do_elu=do_elu, heads=heads),
        grid=(NPAD // blk,),
        in_specs=[
            pl.BlockSpec((blk, 128), lambda i: (i, 0)),
            pl.BlockSpec((blk, 128), lambda i: (i, 0)),
            pl.BlockSpec((blk, heads), lambda i: (i, 0)),
            pl.BlockSpec((blk, heads), lambda i: (i, 0)),
            pl.BlockSpec((1, 128), lambda i: (0, 0)),
            pl.BlockSpec((blk, 128), lambda i: (i, 0)),
        ],
        out_specs=pl.BlockSpec((blk, 128), lambda i: (i, 0)),
        out_shape=jax.ShapeDtypeStruct((NPAD, 128), jnp.float32),
    )(f0, f1, d0, d1, b.reshape(1, 128), skip)


def _gat_layer(xp, srcm, dstm, zero, W, b, att_src, att_dst, heads, out_ch,
               do_elu):
    cat, a_d = _prep(xp, W, att_src, att_dst, heads, out_ch)
    acc = _sc_edge_kernel(heads)(cat, a_d, srcm, dstm, zero)
    f0 = acc[0, :, :128]
    f1 = acc[1, :, :128]
    d0 = acc[0, :, 128:128 + heads]
    d1 = acc[1, :, 128:128 + heads]
    return _finish(f0, f1, d0, d1, b, xp, do_elu=do_elu, heads=heads)


def kernel(x, edge_index, W1, b1, att_src1, att_dst1, W2, b2, att_src2,
           att_dst2):
    N = x.shape[0]
    loop = jnp.arange(N, dtype=jnp.int32)
    src = jnp.concatenate([edge_index[0].astype(jnp.int32), loop])
    dst = jnp.concatenate([edge_index[1].astype(jnp.int32), loop])
    npad_e = EPAD - src.shape[0]
    src = jnp.concatenate([src, jnp.full((npad_e,), N, jnp.int32)])
    dst = jnp.concatenate([dst, jnp.full((npad_e,), N, jnp.int32)])
    srcm = src.reshape(32, NBLK, IB, K)
    dstm = dst.reshape(32, NBLK, IB, K)
    zero = jnp.zeros((NPAD, 144), jnp.float32)
    xp = jnp.pad(x, ((0, NPAD - N), (0, 0)))
    h = _gat_layer(xp, srcm, dstm, zero, W1, b1, att_src1, att_dst1, 8, 16,
                   True)
    h = _gat_layer(h, srcm, dstm, zero, W2, b2, att_src2, att_dst2, 1, 128,
                   False)
    return h[:N]


# speedup vs baseline: 38.8141x; 1.1549x over previous
import dataclasses
import functools

import jax
import jax.numpy as jnp
from jax import lax
from jax.experimental import pallas as pl
from jax.experimental.pallas import tpu as pltpu
from jax.experimental.pallas import tpu_sc as plsc

NPAD = 10112
K = 64
IB = 2
CHUNKS = 164
NBLK = CHUNKS // IB
PER_TILE = CHUNKS * K
EPAD = 32 * PER_TILE
ROWS_PER_TILE = NPAD // 16


def _prep_kernel(x_ref, wt_ref, gs_ref, gd_ref, pm_ref, as_ref, fb_ref,
                 ad_ref):
    h = jnp.dot(x_ref[...], wt_ref[...], preferred_element_type=jnp.float32)
    hp = jnp.dot(h, pm_ref[...], preferred_element_type=jnp.float32)
    fb_ref[...] = hp.astype(jnp.bfloat16)
    as_ref[...] = jnp.dot(h, gs_ref[...], preferred_element_type=jnp.float32)
    ad_ref[...] = jnp.dot(h, gd_ref[...], preferred_element_type=jnp.float32)


def _perm():
    perm = []
    for jj in range(4):
        for t in range(16):
            perm.append(32 * jj + t)
            perm.append(32 * jj + 16 + t)
    return perm


def _prep(xp, W, att_src, att_dst, heads, out_ch):
    eye = jnp.repeat(jnp.eye(heads, 16, dtype=jnp.float32), out_ch, axis=0)
    gs = eye * att_src.reshape(128, 1)
    gd = eye * att_dst.reshape(128, 1)
    pm = jnp.zeros((128, 128), jnp.float32).at[jnp.array(_perm()),
                                               jnp.arange(128)].set(1.0)
    blk = 128
    a_s, fbf, a_d = pl.pallas_call(
        _prep_kernel,
        grid=(NPAD // blk,),
        in_specs=[
            pl.BlockSpec((blk, 128), lambda i: (i, 0)),
            pl.BlockSpec((128, 128), lambda i: (0, 0)),
            pl.BlockSpec((128, 16), lambda i: (0, 0)),
            pl.BlockSpec((128, 16), lambda i: (0, 0)),
            pl.BlockSpec((128, 128), lambda i: (0, 0)),
        ],
        out_specs=[
            pl.BlockSpec((blk, 16), lambda i: (i, 0)),
            pl.BlockSpec((blk, 128), lambda i: (i, 0)),
            pl.BlockSpec((blk, 16), lambda i: (i, 0)),
        ],
        out_shape=[
            jax.ShapeDtypeStruct((NPAD, 16), jnp.float32),
            jax.ShapeDtypeStruct((NPAD, 128), jnp.bfloat16),
            jax.ShapeDtypeStruct((NPAD, 16), jnp.float32),
        ],
    )(xp, W.T, gs, gd, pm)
    packed = jax.lax.bitcast_convert_type(
        fbf.reshape(NPAD, 64, 2), jnp.float32)
    cat = jnp.concatenate([a_s, packed], axis=1)
    return cat, a_d


def _sc_compiler_params():
    cp = pltpu.CompilerParams(use_tc_tiling_on_sc=False)
    if "needs_layout_passes" in pltpu.CompilerParams.__dataclass_fields__:
        cp = dataclasses.replace(cp, needs_layout_passes=False)
    return cp


def _sc_edge_kernel(heads):
    mesh = plsc.VectorSubcoreMesh(core_axis_name="c", subcore_axis_name="s")

    @functools.partial(
        pl.kernel,
        out_type=jax.ShapeDtypeStruct((2, NPAD, 144), jnp.float32),
        mesh=mesh,
        compiler_params=_sc_compiler_params(),
        scratch_types=[
            pltpu.VMEM((IB, K), jnp.int32),
            pltpu.VMEM((IB, K), jnp.int32),
            pltpu.VMEM((IB, K), jnp.int32),
            pltpu.VMEM((IB, K), jnp.int32),
            pltpu.VMEM((K, 80), jnp.float32),
            pltpu.VMEM((K, 80), jnp.float32),
            pltpu.VMEM((K, 16), jnp.float32),
            pltpu.VMEM((K, 16), jnp.float32),
            pltpu.VMEM((K, 144), jnp.float32),
            pltpu.VMEM((K, 144), jnp.float32),
            pltpu.VMEM((K,), jnp.int32),
            pltpu.VMEM((K,), jnp.int32),
            pltpu.VMEM_SHARED((NPAD, 144), jnp.float32),
            pltpu.SemaphoreType.DMA,
            pltpu.SemaphoreType.DMA,
            pltpu.SemaphoreType.DMA,
            pltpu.SemaphoreType.DMA,
            pltpu.SemaphoreType.DMA,
            pltpu.SemaphoreType.DMA,
            pltpu.SemaphoreType.DMA,
            pltpu.SemaphoreType.DMA,
        ],
    )
    def k(cat_hbm, ad_hbm, src_hbm, dst_hbm, zero_hbm, out_hbm,
          si0, si1, di0, di1, ct0, ct1, ad0, ad1, rw0, rw1, sd0, sd1, acc_sh,
          ix0, ix1, cs0, cs1, as0, as1, ss0, ss1):
        c = lax.axis_index("c")
        s = lax.axis_index("s")
        w = c * 16 + s
        si = (si0, si1)
        di = (di0, di1)
        ct = (ct0, ct1)
        ad = (ad0, ad1)
        rw = (rw0, rw1)
        sd = (sd0, sd1)
        ixs = (ix0, ix1)
        css = (cs0, cs1)
        ass = (as0, as1)
        sss = (ss0, ss1)

        def idx_start(blk, ib):
            pltpu.make_async_copy(src_hbm.at[w, blk], si[ib], ixs[ib]).start()
            pltpu.make_async_copy(dst_hbm.at[w, blk], di[ib], ixs[ib]).start()

        def idx_wait(blk, ib):
            pltpu.make_async_copy(src_hbm.at[w, blk], si[ib], ixs[ib]).wait()
            pltpu.make_async_copy(dst_hbm.at[w, blk], di[ib], ixs[ib]).wait()

        def gath_start(ib, r, b):
            pltpu.make_async_copy(cat_hbm.at[si[ib].at[r]], ct[b],
                                  css[b]).start()
            pltpu.make_async_copy(ad_hbm.at[di[ib].at[r]], ad[b],
                                  ass[b]).start()

        def gath_wait(ib, r, b):
            pltpu.make_async_copy(cat_hbm.at[si[ib].at[r]], ct[b],
                                  css[b]).wait()
            pltpu.make_async_copy(ad_hbm.at[di[ib].at[r]], ad[b],
                                  ass[b]).wait()

        def scat_start(b):
            pltpu.make_async_copy(rw[b], acc_sh.at[sd[b]],
                                  sss[b]).start(add=True)

        def scat_wait(b):
            pltpu.make_async_copy(rw[b], acc_sh.at[sd[b]], sss[b]).wait()

        def compute(b):
            ct_v, ad_v, rw_v = ct[b], ad[b], rw[b]

            @pl.loop(0, K, unroll=4)
            def _edge(e):
                va = ct_v[e, pl.ds(0, 16)] + ad_v[e, :]
                va = jnp.maximum(va, 0.2 * va)
                wv = jnp.exp(va)
                rw_v[e, pl.ds(128, 16)] = wv
                for jj in range(4):
                    p = ct_v[e, pl.ds(16 + 16 * jj, 16)]
                    pb = plsc.bitcast(p, jnp.bfloat16)
                    fa, fb = plsc.unpack(pb,
                                         format=plsc.PackFormat.INTERLEAVED)
                    la = 2 * jj if heads == 8 else 0
                    lb = 2 * jj + 1 if heads == 8 else 0
                    wa = wv.at[jnp.full((16,), la, jnp.int32)].get(
                        mode="promise_in_bounds")
                    wb = wv.at[jnp.full((16,), lb, jnp.int32)].get(
                        mode="promise_in_bounds")
                    rw_v[e, pl.ds(32 * jj, 16)] = fa * wa
                    rw_v[e, pl.ds(32 * jj + 16, 16)] = fb * wb

        pltpu.sync_copy(zero_hbm.at[pl.ds(s * ROWS_PER_TILE, ROWS_PER_TILE)],
                        acc_sh.at[pl.ds(s * ROWS_PER_TILE, ROWS_PER_TILE)])
        plsc.subcore_barrier()

        idx_start(0, 0)
        idx_start(1, 1)
        idx_wait(0, 0)
        gath_start(0, 0, 0)

        @pl.loop(0, CHUNKS, step=4)
        def _chunk(g4):
            for u in range(4):
                g = g4 + u
                ib = (u // IB) % 2
                b = u % 2
                nib = ((u + 1) // IB) % 2
                nb = (u + 1) % 2

                gath_wait(ib, b, b)

                @pl.when(g >= 2)
                def _():
                    scat_wait(b)

                for t in range(K // 16):
                    sd[b][pl.ds(16 * t, 16)] = di[ib][b, pl.ds(16 * t, 16)]

                if u % 2 == 1:
                    @pl.when(g // IB + 2 < NBLK)
                    def _():
                        pltpu.make_async_copy(
                            src_hbm.at[w, g // IB + 2], si[ib],
                            ixs[ib]).start()
                        pltpu.make_async_copy(
                            dst_hbm.at[w, g // IB + 2], di[ib],
                            ixs[ib]).start()

                    @pl.when(g + 1 < CHUNKS)
                    def _():
                        pltpu.make_async_copy(
                            src_hbm.at[w, (g + 1) // IB], si[nib],
                            ixs[nib]).wait()
                        pltpu.make_async_copy(
                            dst_hbm.at[w, (g + 1) // IB], di[nib],
                            ixs[nib]).wait()

                @pl.when(g + 1 < CHUNKS)
                def _():
                    gath_start(nib, nb, nb)

                compute(b)
                scat_start(b)

        scat_wait(0)
        scat_wait(1)
        plsc.subcore_barrier()
        pltpu.sync_copy(acc_sh.at[pl.ds(s * ROWS_PER_TILE, ROWS_PER_TILE)],
                        out_hbm.at[c, pl.ds(s * ROWS_PER_TILE, ROWS_PER_TILE)])

    return k


def _finish_kernel(f0_ref, f1_ref, d0_ref, d1_ref, b_ref, skip_ref, o_ref,
                   *, do_elu, heads):
    feat = f0_ref[...] + f1_ref[...]
    denom = d0_ref[...] + d1_ref[...]
    d = jnp.repeat(denom, 128 // heads, axis=1)
    out = feat / (d + 1e-16) + b_ref[...]
    if do_elu:
        out = jnp.where(out > 0, out, jnp.exp(jnp.minimum(out, 0.0)) - 1.0)
    o_ref[...] = out + skip_ref[...]


def _finish(f0, f1, d0, d1, b, skip, *, do_elu, heads):
    blk = 128
    return pl.pallas_call(
        functools.partial(_finish_kernel, do_elu=do_elu, heads=heads),
        grid=(NPAD // blk,),
        in_specs=[
            pl.BlockSpec((blk, 128), lambda i: (i, 0)),
            pl.BlockSpec((blk, 128), lambda i: (i, 0)),
            pl.BlockSpec((blk, heads), lambda i: (i, 0)),
            pl.BlockSpec((blk, heads), lambda i: (i, 0)),
            pl.BlockSpec((1, 128), lambda i: (0, 0)),
            pl.BlockSpec((blk, 128), lambda i: (i, 0)),
        ],
        out_specs=pl.BlockSpec((blk, 128), lambda i: (i, 0)),
        out_shape=jax.ShapeDtypeStruct((NPAD, 128), jnp.float32),
    )(f0, f1, d0, d1, b.reshape(1, 128), skip)


def _gat_layer(xp, srcm, dstm, zero, W, b, att_src, att_dst, heads, out_ch,
               do_elu):
    cat, a_d = _prep(xp, W, att_src, att_dst, heads, out_ch)
    acc = _sc_edge_kernel(heads)(cat, a_d, srcm, dstm, zero)
    f0 = acc[0, :, :128]
    f1 = acc[1, :, :128]
    d0 = acc[0, :, 128:128 + heads]
    d1 = acc[1, :, 128:128 + heads]
    return _finish(f0, f1, d0, d1, b, xp, do_elu=do_elu, heads=heads)


def kernel(x, edge_index, W1, b1, att_src1, att_dst1, W2, b2, att_src2,
           att_dst2):
    N = x.shape[0]
    loop = jnp.arange(N, dtype=jnp.int32)
    src = jnp.concatenate([edge_index[0].astype(jnp.int32), loop])
    dst = jnp.concatenate([edge_index[1].astype(jnp.int32), loop])
    npad_e = EPAD - src.shape[0]
    src = jnp.concatenate([src, jnp.full((npad_e,), N, jnp.int32)])
    dst = jnp.concatenate([dst, jnp.full((npad_e,), N, jnp.int32)])
    srcm = src.reshape(32, NBLK, IB, K)
    dstm = dst.reshape(32, NBLK, IB, K)
    zero = jnp.zeros((NPAD, 144), jnp.float32)
    xp = jnp.pad(x, ((0, NPAD - N), (0, 0)))
    h = _gat_layer(xp, srcm, dstm, zero, W1, b1, att_src1, att_dst1, 8, 16,
                   True)
    h = _gat_layer(h, srcm, dstm, zero, W2, b2, att_src2, att_dst2, 1, 128,
                   False)
    return h[:N]
